# baseline (speedup 1.0000x reference)
import functools

import jax
import jax.numpy as jnp
from jax import lax
from jax.experimental import pallas as pl
from jax.experimental.pallas import tpu as pltpu
from jax.experimental.pallas import tpu_sc as plsc

N = 10000
E = 320000
D_FEAT = 128
DH = 64
NS = 16
NC = 2
E_PER_TILE = E // NS
CHUNK = 128
NITER = 160
E_PAD = NITER * CHUNK
PAD = E_PAD - E_PER_TILE
STRIPE = N // NS
NRING = 3
LROWS = 640
HB = LROWS // NS


def _make_sc_lap(with_deg):
    mesh = plsc.VectorSubcoreMesh(core_axis_name="c", subcore_axis_name="s")
    out_type = [jax.ShapeDtypeStruct((NC, NS, STRIPE, DH), jnp.bfloat16)]
    if with_deg:
        out_type.append(jax.ShapeDtypeStruct((NS, HB, 16), jnp.float32))

    scratch = [
        pltpu.VMEM((NITER, CHUNK), jnp.int32),
        pltpu.VMEM((NITER, CHUNK), jnp.int32),
        pltpu.VMEM((NRING, CHUNK, DH), jnp.bfloat16),
        pltpu.VMEM_SHARED((N + 16, DH), jnp.bfloat16),
        [pltpu.SemaphoreType.DMA] * NRING,
        [pltpu.SemaphoreType.DMA] * NRING,
    ]
    if with_deg:
        scratch += [
            pltpu.VMEM((LROWS, 16), jnp.float32),
            pltpu.VMEM((5, CHUNK), jnp.int32),
            pltpu.VMEM_SHARED((LROWS, 16), jnp.float32),
        ]

    @functools.partial(pl.kernel, out_type=out_type, mesh=mesh,
                       compiler_params=pltpu.CompilerParams(
                           use_tc_tiling_on_sc=False,
                           needs_layout_passes=False),
                       scratch_types=scratch)
    def lap(table, src_adj, dst_r, *refs):
        if with_deg:
            (out, deg_out, sidx, didx, bfr, agg, semg, sems,
             ldeg, idv, degsp) = refs
        else:
            out, sidx, didx, bfr, agg, semg, sems = refs
        c = lax.axis_index("c")
        s = lax.axis_index("s")

        pltpu.sync_copy(src_adj.at[c, s], sidx)
        pltpu.sync_copy(dst_r.at[s], didx)

        zv = jnp.zeros((16,), jnp.float32)
        zvb = jnp.zeros((32,), jnp.bfloat16)

        def zrow(i, carry):
            for j in range(DH // 32):
                bfr[0, i, pl.ds(j * 32, 32)] = zvb
            return carry

        lax.fori_loop(0, CHUNK, zrow, 0)
        for k in range(STRIPE // CHUNK):
            pltpu.sync_copy(bfr.at[0],
                            agg.at[pl.ds(s * STRIPE + k * CHUNK, CHUNK)])
        rem = STRIPE % CHUNK
        pltpu.sync_copy(
            bfr.at[0, pl.ds(0, rem)],
            agg.at[pl.ds(s * STRIPE + (STRIPE // CHUNK) * CHUNK, rem)])

        @pl.when(s == 0)
        def _():
            pltpu.sync_copy(bfr.at[0, pl.ds(0, 16)], agg.at[pl.ds(N, 16)])

        if with_deg:
            def zhrow(i, carry):
                ldeg[i, pl.ds(0, 16)] = zv
                return carry

            lax.fori_loop(0, LROWS, zhrow, 0)
            pltpu.sync_copy(ldeg.at[pl.ds(0, HB)],
                            degsp.at[pl.ds(s * HB, HB)])
            iota = lax.iota(jnp.int32, 16)
            for k in range(5):
                for j in range(CHUNK // 16):
                    idv[k, pl.ds(16 * j, 16)] = iota + (CHUNK * k + 16 * j)

        plsc.subcore_barrier()

        def g_start(b, i):
            pltpu.async_copy(table.at[sidx.at[i]], bfr.at[b], semg[b])

        def g_wait(b, i):
            pltpu.make_async_copy(table.at[sidx.at[i]], bfr.at[b],
                                  semg[b]).wait()

        def s_start(b, i):
            pltpu.async_copy(bfr.at[b], agg.at[didx.at[i]], sems[b],
                             add=True)

        def s_wait(b, i):
            pltpu.make_async_copy(bfr.at[b], agg.at[didx.at[i]],
                                  sems[b]).wait()

        onesv = jnp.ones((16,), jnp.float32)

        def hist(i):
            if with_deg:
                for j in range(CHUNK // 16):
                    nv = didx[i, pl.ds(16 * j, 16)]
                    row = lax.shift_right_logical(nv, 4)
                    col = jnp.bitwise_and(nv, 15)
                    plsc.addupdate_scatter(ldeg, [row, col], onesv)

        def step(i, b, do_swait=True, do_gstart=True):
            hist(i)
            if do_swait:
                s_wait((b + 2) % NRING, i - 1)
            g_wait(b, i)
            s_start(b, i)
            if do_gstart:
                g_start((b + 2) % NRING, i + 2)

        g_start(0, 0)
        g_start(1, 1)
        step(0, 0, do_swait=False)

        def body(j, carry):
            for b in range(NRING):
                step(1 + NRING * j + b, (1 + b) % NRING)
            return carry

        lax.fori_loop(0, (NITER - 4) // NRING, body, 0)

        step(NITER - 3, (NITER - 3) % NRING)
        step(NITER - 2, (NITER - 2) % NRING, do_gstart=False)
        step(NITER - 1, (NITER - 1) % NRING, do_gstart=False)
        s_wait((NITER - 1) % NRING, NITER - 1)

        if with_deg:
            plsc.subcore_barrier()
            for k in range(5):
                pltpu.sync_copy(ldeg.at[pl.ds(CHUNK * k, CHUNK)],
                                degsp.at[idv.at[k]], add=True)
            plsc.subcore_barrier()

            @pl.when(c == 0)
            def _():
                pltpu.sync_copy(degsp.at[pl.ds(s * HB, HB)], deg_out.at[s])

        plsc.subcore_barrier()
        pltpu.sync_copy(agg.at[pl.ds(s * STRIPE, STRIPE)], out.at[c, s])

    return lap


_sc_lap_deg = _make_sc_lap(True)
_sc_lap = _make_sc_lap(False)


def _tc_matmul(x, W):
    B = 1000

    def body(x_ref, w_ref, m2_ref, tb_ref):
        xv = x_ref[...]
        m2v = jnp.dot(xv, w_ref[...], preferred_element_type=jnp.float32)
        m2_ref[...] = m2v
        tbv = m2v.astype(jnp.bfloat16)
        tb_ref[0] = tbv[:, :DH]
        tb_ref[1] = tbv[:, DH:]

    return pl.pallas_call(
        body,
        grid=(N // B,),
        in_specs=[
            pl.BlockSpec((B, D_FEAT), lambda i: (i, 0)),
            pl.BlockSpec((D_FEAT, D_FEAT), lambda i: (0, 0)),
        ],
        out_specs=[
            pl.BlockSpec((B, D_FEAT), lambda i: (i, 0)),
            pl.BlockSpec((NC, B, DH), lambda i: (0, i, 0)),
        ],
        out_shape=[
            jax.ShapeDtypeStruct((N, D_FEAT), jnp.float32),
            jax.ShapeDtypeStruct((NC, N, DH), jnp.bfloat16),
        ],
    )(x, W)


def _tc_mid(m2, agg1, deg):
    B = 1000

    def body(m2_ref, a_ref, deg_ref, m1_ref, t2_ref, dv_ref):
        m2v = m2_ref[...]
        aggcat = jnp.concatenate([a_ref[0], a_ref[1]],
                                 axis=1).astype(jnp.float32)
        degv = deg_ref[...]
        m1 = degv * m2v - aggcat
        dinv = jnp.where(degv > 0.0, 1.0 / jnp.sqrt(jnp.maximum(degv, 1.0)),
                         0.0)
        hs = m1 * dinv
        hsp = hs.astype(jnp.bfloat16)
        m1_ref[...] = m1
        t2_ref[0] = hsp[:, :DH]
        t2_ref[1] = hsp[:, DH:]
        dv_ref[...] = dinv

    return pl.pallas_call(
        body,
        grid=(N // B,),
        in_specs=[
            pl.BlockSpec((B, D_FEAT), lambda i: (i, 0)),
            pl.BlockSpec((NC, B, DH), lambda i: (0, i, 0)),
            pl.BlockSpec((B, 1), lambda i: (i, 0)),
        ],
        out_specs=[
            pl.BlockSpec((B, D_FEAT), lambda i: (i, 0)),
            pl.BlockSpec((NC, B, DH), lambda i: (0, i, 0)),
            pl.BlockSpec((B, 1), lambda i: (i, 0)),
        ],
        out_shape=[
            jax.ShapeDtypeStruct((N, D_FEAT), jnp.float32),
            jax.ShapeDtypeStruct((NC, N, DH), jnp.bfloat16),
            jax.ShapeDtypeStruct((N, 1), jnp.float32),
        ],
    )(m2, agg1, deg)


def _tc_final(m2, m1, dv, agg2):
    B = 1000

    def body(m2_ref, m1_ref, dv_ref, a_ref, o_ref):
        aggcat = jnp.concatenate([a_ref[0], a_ref[1]],
                                 axis=1).astype(jnp.float32)
        o_ref[...] = jnp.maximum(
            m2_ref[...] + 3.0 * m1_ref[...] - 2.0 * dv_ref[...] * aggcat, 0.0)

    return pl.pallas_call(
        body,
        grid=(N // B,),
        in_specs=[
            pl.BlockSpec((B, D_FEAT), lambda i: (i, 0)),
            pl.BlockSpec((B, D_FEAT), lambda i: (i, 0)),
            pl.BlockSpec((B, 1), lambda i: (i, 0)),
            pl.BlockSpec((NC, B, DH), lambda i: (0, i, 0)),
        ],
        out_specs=pl.BlockSpec((B, D_FEAT), lambda i: (i, 0)),
        out_shape=jax.ShapeDtypeStruct((N, D_FEAT), jnp.float32),
    )(m2, m1, dv, agg2)


def kernel(x, edge_index, W):
    src = edge_index[0].astype(jnp.int32)
    dst = edge_index[1].astype(jnp.int32)

    src_r = src.reshape(NS, E_PER_TILE)
    offs = (jnp.arange(NC, dtype=jnp.int32) * N)[:, None, None]
    src_adj = jnp.concatenate(
        [src_r[None] + offs,
         jnp.zeros((NC, NS, PAD), jnp.int32)],
        axis=2).reshape(NC, NS, NITER, CHUNK)
    dst_p = jnp.concatenate(
        [dst.reshape(NS, E_PER_TILE),
         jnp.full((NS, PAD), N, jnp.int32)],
        axis=1).reshape(NS, NITER, CHUNK)

    m2, tb1 = _tc_matmul(x, W)

    agg1, deg_t = _sc_lap_deg(tb1.reshape(NC * N, DH), src_adj, dst_p)
    agg1 = agg1.reshape(NC, N, DH)
    deg = deg_t.reshape(NS * HB * 16)[:N].reshape(N, 1)

    m1, t2, dv = _tc_mid(m2, agg1, deg)

    agg2, = _sc_lap(t2.reshape(NC * N, DH), src_adj, dst_p)
    agg2 = agg2.reshape(NC, N, DH)

    return _tc_final(m2, m1, dv, agg2)


# speedup vs baseline: 1.3612x; 1.3612x over previous
import functools

import jax
import jax.numpy as jnp
from jax import lax
from jax.experimental import pallas as pl
from jax.experimental.pallas import tpu as pltpu
from jax.experimental.pallas import tpu_sc as plsc

N = 10000
E = 320000
D_FEAT = 128
DH = 64
NS = 16
NC = 2
E_PER_TILE = E // NS
CHUNK = 128
NITER = 160
E_PAD = NITER * CHUNK
PAD = E_PAD - E_PER_TILE
STRIPE = N // NS
NRING = 3
LROWS = 640
HB = LROWS // NS


def _make_sc_lap(with_deg):
    mesh = plsc.VectorSubcoreMesh(core_axis_name="c", subcore_axis_name="s")
    out_type = [jax.ShapeDtypeStruct((NC, NS, STRIPE, DH), jnp.bfloat16)]
    if with_deg:
        out_type.append(jax.ShapeDtypeStruct((NS, HB, 16), jnp.float32))

    scratch = [
        pltpu.VMEM((NITER, CHUNK), jnp.int32),
        pltpu.VMEM((NITER, CHUNK), jnp.int32),
        pltpu.VMEM((NITER, CHUNK), jnp.int32),
        pltpu.VMEM((NRING, CHUNK, DH), jnp.bfloat16),
        pltpu.VMEM_SHARED((N + 16, DH), jnp.bfloat16),
        pltpu.VMEM_SHARED((N, DH), jnp.bfloat16),
        [pltpu.SemaphoreType.DMA] * NRING,
        [pltpu.SemaphoreType.DMA] * NRING,
    ]
    if with_deg:
        scratch += [
            pltpu.VMEM((LROWS, 16), jnp.float32),
            pltpu.VMEM((5, CHUNK), jnp.int32),
            pltpu.VMEM_SHARED((LROWS, 16), jnp.float32),
        ]

    @functools.partial(pl.kernel, out_type=out_type, mesh=mesh,
                       compiler_params=pltpu.CompilerParams(
                           use_tc_tiling_on_sc=False,
                           needs_layout_passes=False),
                       scratch_types=scratch)
    def lap(table, src_adj, src_r, dst_r, *refs):
        if with_deg:
            (out, deg_out, sidx, sidx2, didx, bfr, agg, tsp, semg, sems,
             ldeg, idv, degsp) = refs
        else:
            out, sidx, sidx2, didx, bfr, agg, tsp, semg, sems = refs
        c = lax.axis_index("c")
        s = lax.axis_index("s")

        pltpu.sync_copy(src_adj.at[c, s], sidx)
        pltpu.sync_copy(src_r.at[s], sidx2)
        pltpu.sync_copy(dst_r.at[s], didx)
        pltpu.sync_copy(table.at[pl.ds(c * N + s * STRIPE, STRIPE)],
                        tsp.at[pl.ds(s * STRIPE, STRIPE)])

        zv = jnp.zeros((16,), jnp.float32)
        zvb = jnp.zeros((32,), jnp.bfloat16)

        def zrow(i, carry):
            for j in range(DH // 32):
                bfr[0, i, pl.ds(j * 32, 32)] = zvb
            return carry

        lax.fori_loop(0, CHUNK, zrow, 0)
        for k in range(STRIPE // CHUNK):
            pltpu.sync_copy(bfr.at[0],
                            agg.at[pl.ds(s * STRIPE + k * CHUNK, CHUNK)])
        rem = STRIPE % CHUNK
        pltpu.sync_copy(
            bfr.at[0, pl.ds(0, rem)],
            agg.at[pl.ds(s * STRIPE + (STRIPE // CHUNK) * CHUNK, rem)])

        @pl.when(s == 0)
        def _():
            pltpu.sync_copy(bfr.at[0, pl.ds(0, 16)], agg.at[pl.ds(N, 16)])

        if with_deg:
            def zhrow(i, carry):
                ldeg[i, pl.ds(0, 16)] = zv
                return carry

            lax.fori_loop(0, LROWS, zhrow, 0)
            pltpu.sync_copy(ldeg.at[pl.ds(0, HB)],
                            degsp.at[pl.ds(s * HB, HB)])
            iota = lax.iota(jnp.int32, 16)
            for k in range(5):
                for j in range(CHUNK // 16):
                    idv[k, pl.ds(16 * j, 16)] = iota + (CHUNK * k + 16 * j)

        plsc.subcore_barrier()

        def g_start(b, i, sp):
            if sp:
                pltpu.async_copy(tsp.at[sidx2.at[i]], bfr.at[b], semg[b])
            else:
                pltpu.async_copy(table.at[sidx.at[i]], bfr.at[b], semg[b])

        def g_wait(b, i, sp):
            if sp:
                pltpu.make_async_copy(tsp.at[sidx2.at[i]], bfr.at[b],
                                      semg[b]).wait()
            else:
                pltpu.make_async_copy(table.at[sidx.at[i]], bfr.at[b],
                                      semg[b]).wait()

        def s_start(b, i):
            pltpu.async_copy(bfr.at[b], agg.at[didx.at[i]], sems[b],
                             add=True)

        def s_wait(b, i):
            pltpu.make_async_copy(bfr.at[b], agg.at[didx.at[i]],
                                  sems[b]).wait()

        onesv = jnp.ones((16,), jnp.float32)

        def hist(i):
            if with_deg:
                for j in range(CHUNK // 16):
                    nv = didx[i, pl.ds(16 * j, 16)]
                    row = lax.shift_right_logical(nv, 4)
                    col = jnp.bitwise_and(nv, 15)
                    plsc.addupdate_scatter(ldeg, [row, col], onesv)

        def step(i, b, par, do_swait=True, do_gstart=True):
            hist(i)
            if do_swait:
                s_wait((b + 2) % NRING, i - 1)
            g_wait(b, i, par)
            s_start(b, i)
            if do_gstart:
                g_start((b + 2) % NRING, i + 2, (par + 2) % 2)

        g_start(0, 0, 0)
        g_start(1, 1, 1)
        step(0, 0, 0, do_swait=False)

        def body(j, carry):
            for k in range(2 * NRING):
                i = 1 + 2 * NRING * j + k
                step(i, (1 + k) % NRING, (1 + k) % 2)
            return carry

        lax.fori_loop(0, (NITER - 4) // (2 * NRING), body, 0)

        step(NITER - 3, (NITER - 3) % NRING, (NITER - 3) % 2)
        step(NITER - 2, (NITER - 2) % NRING, (NITER - 2) % 2,
             do_gstart=False)
        step(NITER - 1, (NITER - 1) % NRING, (NITER - 1) % 2,
             do_gstart=False)
        s_wait((NITER - 1) % NRING, NITER - 1)

        if with_deg:
            plsc.subcore_barrier()
            for k in range(5):
                pltpu.sync_copy(ldeg.at[pl.ds(CHUNK * k, CHUNK)],
                                degsp.at[idv.at[k]], add=True)
            plsc.subcore_barrier()

            @pl.when(c == 0)
            def _():
                pltpu.sync_copy(degsp.at[pl.ds(s * HB, HB)], deg_out.at[s])

        plsc.subcore_barrier()
        pltpu.sync_copy(agg.at[pl.ds(s * STRIPE, STRIPE)], out.at[c, s])

    return lap


_sc_lap_deg = _make_sc_lap(True)
_sc_lap = _make_sc_lap(False)


def _tc_matmul(x, W):
    B = 1000

    def body(x_ref, w_ref, m2_ref, tb_ref):
        xv = x_ref[...]
        m2v = jnp.dot(xv, w_ref[...], preferred_element_type=jnp.float32)
        m2_ref[...] = m2v
        tbv = m2v.astype(jnp.bfloat16)
        tb_ref[0] = tbv[:, :DH]
        tb_ref[1] = tbv[:, DH:]

    return pl.pallas_call(
        body,
        grid=(N // B,),
        in_specs=[
            pl.BlockSpec((B, D_FEAT), lambda i: (i, 0)),
            pl.BlockSpec((D_FEAT, D_FEAT), lambda i: (0, 0)),
        ],
        out_specs=[
            pl.BlockSpec((B, D_FEAT), lambda i: (i, 0)),
            pl.BlockSpec((NC, B, DH), lambda i: (0, i, 0)),
        ],
        out_shape=[
            jax.ShapeDtypeStruct((N, D_FEAT), jnp.float32),
            jax.ShapeDtypeStruct((NC, N, DH), jnp.bfloat16),
        ],
    )(x, W)


def _tc_mid(m2, agg1, deg):
    B = 1000

    def body(m2_ref, a_ref, deg_ref, m1_ref, t2_ref, dv_ref):
        m2v = m2_ref[...]
        aggcat = jnp.concatenate([a_ref[0], a_ref[1]],
                                 axis=1).astype(jnp.float32)
        degv = deg_ref[...]
        m1 = degv * m2v - aggcat
        dinv = jnp.where(degv > 0.0, 1.0 / jnp.sqrt(jnp.maximum(degv, 1.0)),
                         0.0)
        hs = m1 * dinv
        hsp = hs.astype(jnp.bfloat16)
        m1_ref[...] = m1
        t2_ref[0] = hsp[:, :DH]
        t2_ref[1] = hsp[:, DH:]
        dv_ref[...] = dinv

    return pl.pallas_call(
        body,
        grid=(N // B,),
        in_specs=[
            pl.BlockSpec((B, D_FEAT), lambda i: (i, 0)),
            pl.BlockSpec((NC, B, DH), lambda i: (0, i, 0)),
            pl.BlockSpec((B, 1), lambda i: (i, 0)),
        ],
        out_specs=[
            pl.BlockSpec((B, D_FEAT), lambda i: (i, 0)),
            pl.BlockSpec((NC, B, DH), lambda i: (0, i, 0)),
            pl.BlockSpec((B, 1), lambda i: (i, 0)),
        ],
        out_shape=[
            jax.ShapeDtypeStruct((N, D_FEAT), jnp.float32),
            jax.ShapeDtypeStruct((NC, N, DH), jnp.bfloat16),
            jax.ShapeDtypeStruct((N, 1), jnp.float32),
        ],
    )(m2, agg1, deg)


def _tc_final(m2, m1, dv, agg2):
    B = 1000

    def body(m2_ref, m1_ref, dv_ref, a_ref, o_ref):
        aggcat = jnp.concatenate([a_ref[0], a_ref[1]],
                                 axis=1).astype(jnp.float32)
        o_ref[...] = jnp.maximum(
            m2_ref[...] + 3.0 * m1_ref[...] - 2.0 * dv_ref[...] * aggcat, 0.0)

    return pl.pallas_call(
        body,
        grid=(N // B,),
        in_specs=[
            pl.BlockSpec((B, D_FEAT), lambda i: (i, 0)),
            pl.BlockSpec((B, D_FEAT), lambda i: (i, 0)),
            pl.BlockSpec((B, 1), lambda i: (i, 0)),
            pl.BlockSpec((NC, B, DH), lambda i: (0, i, 0)),
        ],
        out_specs=pl.BlockSpec((B, D_FEAT), lambda i: (i, 0)),
        out_shape=jax.ShapeDtypeStruct((N, D_FEAT), jnp.float32),
    )(m2, m1, dv, agg2)


def kernel(x, edge_index, W):
    src = edge_index[0].astype(jnp.int32)
    dst = edge_index[1].astype(jnp.int32)

    src_r = src.reshape(NS, E_PER_TILE)
    offs = (jnp.arange(NC, dtype=jnp.int32) * N)[:, None, None]
    src_adj = jnp.concatenate(
        [src_r[None] + offs,
         jnp.zeros((NC, NS, PAD), jnp.int32)],
        axis=2).reshape(NC, NS, NITER, CHUNK)
    src_u = jnp.concatenate(
        [src_r, jnp.zeros((NS, PAD), jnp.int32)],
        axis=1).reshape(NS, NITER, CHUNK)
    dst_p = jnp.concatenate(
        [dst.reshape(NS, E_PER_TILE),
         jnp.full((NS, PAD), N, jnp.int32)],
        axis=1).reshape(NS, NITER, CHUNK)

    m2, tb1 = _tc_matmul(x, W)

    agg1, deg_t = _sc_lap_deg(tb1.reshape(NC * N, DH), src_adj, src_u, dst_p)
    agg1 = agg1.reshape(NC, N, DH)
    deg = deg_t.reshape(NS * HB * 16)[:N].reshape(N, 1)

    m1, t2, dv = _tc_mid(m2, agg1, deg)

    agg2, = _sc_lap(t2.reshape(NC * N, DH), src_adj, src_u, dst_p)
    agg2 = agg2.reshape(NC, N, DH)

    return _tc_final(m2, m1, dv, agg2)


# speedup vs baseline: 1.3940x; 1.0241x over previous
import functools

import jax
import jax.numpy as jnp
from jax import lax
from jax.experimental import pallas as pl
from jax.experimental.pallas import tpu as pltpu
from jax.experimental.pallas import tpu_sc as plsc

N = 10000
E = 320000
D_FEAT = 128
DH = 64
NS = 16
NC = 2
E_PER_TILE = E // NS
CHUNK = 128
NITER = 160
E_PAD = NITER * CHUNK
PAD = E_PAD - E_PER_TILE
STRIPE = N // NS
NRING = 3
_PATH = (0, 1, 1, 0, 1, 1)
LROWS = 640
HB = LROWS // NS


def _make_sc_lap(with_deg):
    mesh = plsc.VectorSubcoreMesh(core_axis_name="c", subcore_axis_name="s")
    out_type = [jax.ShapeDtypeStruct((NC, NS, STRIPE, DH), jnp.bfloat16)]
    if with_deg:
        out_type.append(jax.ShapeDtypeStruct((NS, HB, 16), jnp.float32))

    scratch = [
        pltpu.VMEM((NITER, CHUNK), jnp.int32),
        pltpu.VMEM((NITER, CHUNK), jnp.int32),
        pltpu.VMEM((NITER, CHUNK), jnp.int32),
        pltpu.VMEM((NRING, CHUNK, DH), jnp.bfloat16),
        pltpu.VMEM_SHARED((N + 16, DH), jnp.bfloat16),
        pltpu.VMEM_SHARED((N, DH), jnp.bfloat16),
        [pltpu.SemaphoreType.DMA] * NRING,
        [pltpu.SemaphoreType.DMA] * NRING,
    ]
    if with_deg:
        scratch += [
            pltpu.VMEM((LROWS, 16), jnp.float32),
            pltpu.VMEM((5, CHUNK), jnp.int32),
            pltpu.VMEM_SHARED((LROWS, 16), jnp.float32),
        ]

    @functools.partial(pl.kernel, out_type=out_type, mesh=mesh,
                       compiler_params=pltpu.CompilerParams(
                           use_tc_tiling_on_sc=False,
                           needs_layout_passes=False),
                       scratch_types=scratch)
    def lap(table, src_adj, src_r, dst_r, *refs):
        if with_deg:
            (out, deg_out, sidx, sidx2, didx, bfr, agg, tsp, semg, sems,
             ldeg, idv, degsp) = refs
        else:
            out, sidx, sidx2, didx, bfr, agg, tsp, semg, sems = refs
        c = lax.axis_index("c")
        s = lax.axis_index("s")

        pltpu.sync_copy(src_adj.at[c, s], sidx)
        pltpu.sync_copy(src_r.at[s], sidx2)
        pltpu.sync_copy(dst_r.at[s], didx)
        pltpu.sync_copy(table.at[pl.ds(c * N + s * STRIPE, STRIPE)],
                        tsp.at[pl.ds(s * STRIPE, STRIPE)])

        zv = jnp.zeros((16,), jnp.float32)
        zvb = jnp.zeros((32,), jnp.bfloat16)

        def zrow(i, carry):
            for j in range(DH // 32):
                bfr[0, i, pl.ds(j * 32, 32)] = zvb
            return carry

        lax.fori_loop(0, CHUNK, zrow, 0)
        for k in range(STRIPE // CHUNK):
            pltpu.sync_copy(bfr.at[0],
                            agg.at[pl.ds(s * STRIPE + k * CHUNK, CHUNK)])
        rem = STRIPE % CHUNK
        pltpu.sync_copy(
            bfr.at[0, pl.ds(0, rem)],
            agg.at[pl.ds(s * STRIPE + (STRIPE // CHUNK) * CHUNK, rem)])

        @pl.when(s == 0)
        def _():
            pltpu.sync_copy(bfr.at[0, pl.ds(0, 16)], agg.at[pl.ds(N, 16)])

        if with_deg:
            def zhrow(i, carry):
                ldeg[i, pl.ds(0, 16)] = zv
                return carry

            lax.fori_loop(0, LROWS, zhrow, 0)
            pltpu.sync_copy(ldeg.at[pl.ds(0, HB)],
                            degsp.at[pl.ds(s * HB, HB)])
            iota = lax.iota(jnp.int32, 16)
            for k in range(5):
                for j in range(CHUNK // 16):
                    idv[k, pl.ds(16 * j, 16)] = iota + (CHUNK * k + 16 * j)

        plsc.subcore_barrier()

        def g_start(b, i, sp):
            if sp:
                pltpu.async_copy(tsp.at[sidx2.at[i]], bfr.at[b], semg[b])
            else:
                pltpu.async_copy(table.at[sidx.at[i]], bfr.at[b], semg[b])

        def g_wait(b, i, sp):
            if sp:
                pltpu.make_async_copy(tsp.at[sidx2.at[i]], bfr.at[b],
                                      semg[b]).wait()
            else:
                pltpu.make_async_copy(table.at[sidx.at[i]], bfr.at[b],
                                      semg[b]).wait()

        def s_start(b, i):
            pltpu.async_copy(bfr.at[b], agg.at[didx.at[i]], sems[b],
                             add=True)

        def s_wait(b, i):
            pltpu.make_async_copy(bfr.at[b], agg.at[didx.at[i]],
                                  sems[b]).wait()

        onesv = jnp.ones((16,), jnp.float32)

        def hist(i):
            if with_deg:
                for j in range(CHUNK // 16):
                    nv = didx[i, pl.ds(16 * j, 16)]
                    row = lax.shift_right_logical(nv, 4)
                    col = jnp.bitwise_and(nv, 15)
                    plsc.addupdate_scatter(ldeg, [row, col], onesv)

        def step(i, b, im6, do_swait=True, do_gstart=True):
            par = _PATH[im6]
            hist(i)
            if do_swait:
                s_wait((b + 2) % NRING, i - 1)
            g_wait(b, i, par)
            s_start(b, i)
            if do_gstart:
                g_start((b + 2) % NRING, i + 2, _PATH[(im6 + 2) % 6])

        g_start(0, 0, _PATH[0])
        g_start(1, 1, _PATH[1])
        step(0, 0, 0, do_swait=False)

        def body(j, carry):
            for k in range(2 * NRING):
                i = 1 + 2 * NRING * j + k
                step(i, (1 + k) % NRING, (1 + k) % 6)
            return carry

        lax.fori_loop(0, (NITER - 4) // (2 * NRING), body, 0)

        step(NITER - 3, (NITER - 3) % NRING, (NITER - 3) % 6)
        step(NITER - 2, (NITER - 2) % NRING, (NITER - 2) % 6,
             do_gstart=False)
        step(NITER - 1, (NITER - 1) % NRING, (NITER - 1) % 6,
             do_gstart=False)
        s_wait((NITER - 1) % NRING, NITER - 1)

        if with_deg:
            plsc.subcore_barrier()
            for k in range(5):
                pltpu.sync_copy(ldeg.at[pl.ds(CHUNK * k, CHUNK)],
                                degsp.at[idv.at[k]], add=True)
            plsc.subcore_barrier()

            @pl.when(c == 0)
            def _():
                pltpu.sync_copy(degsp.at[pl.ds(s * HB, HB)], deg_out.at[s])

        plsc.subcore_barrier()
        pltpu.sync_copy(agg.at[pl.ds(s * STRIPE, STRIPE)], out.at[c, s])

    return lap


_sc_lap_deg = _make_sc_lap(True)
_sc_lap = _make_sc_lap(False)


def _tc_matmul(x, W):
    B = 1000

    def body(x_ref, w_ref, m2_ref, tb_ref):
        xv = x_ref[...]
        m2v = jnp.dot(xv, w_ref[...], preferred_element_type=jnp.float32)
        m2_ref[...] = m2v
        tbv = m2v.astype(jnp.bfloat16)
        tb_ref[0] = tbv[:, :DH]
        tb_ref[1] = tbv[:, DH:]

    return pl.pallas_call(
        body,
        grid=(N // B,),
        in_specs=[
            pl.BlockSpec((B, D_FEAT), lambda i: (i, 0)),
            pl.BlockSpec((D_FEAT, D_FEAT), lambda i: (0, 0)),
        ],
        out_specs=[
            pl.BlockSpec((B, D_FEAT), lambda i: (i, 0)),
            pl.BlockSpec((NC, B, DH), lambda i: (0, i, 0)),
        ],
        out_shape=[
            jax.ShapeDtypeStruct((N, D_FEAT), jnp.float32),
            jax.ShapeDtypeStruct((NC, N, DH), jnp.bfloat16),
        ],
    )(x, W)


def _tc_mid(m2, agg1, deg):
    B = 1000

    def body(m2_ref, a_ref, deg_ref, m1_ref, t2_ref, dv_ref):
        m2v = m2_ref[...]
        aggcat = jnp.concatenate([a_ref[0], a_ref[1]],
                                 axis=1).astype(jnp.float32)
        degv = deg_ref[...]
        m1 = degv * m2v - aggcat
        dinv = jnp.where(degv > 0.0, 1.0 / jnp.sqrt(jnp.maximum(degv, 1.0)),
                         0.0)
        hs = m1 * dinv
        hsp = hs.astype(jnp.bfloat16)
        m1_ref[...] = m1
        t2_ref[0] = hsp[:, :DH]
        t2_ref[1] = hsp[:, DH:]
        dv_ref[...] = dinv

    return pl.pallas_call(
        body,
        grid=(N // B,),
        in_specs=[
            pl.BlockSpec((B, D_FEAT), lambda i: (i, 0)),
            pl.BlockSpec((NC, B, DH), lambda i: (0, i, 0)),
            pl.BlockSpec((B, 1), lambda i: (i, 0)),
        ],
        out_specs=[
            pl.BlockSpec((B, D_FEAT), lambda i: (i, 0)),
            pl.BlockSpec((NC, B, DH), lambda i: (0, i, 0)),
            pl.BlockSpec((B, 1), lambda i: (i, 0)),
        ],
        out_shape=[
            jax.ShapeDtypeStruct((N, D_FEAT), jnp.float32),
            jax.ShapeDtypeStruct((NC, N, DH), jnp.bfloat16),
            jax.ShapeDtypeStruct((N, 1), jnp.float32),
        ],
    )(m2, agg1, deg)


def _tc_final(m2, m1, dv, agg2):
    B = 1000

    def body(m2_ref, m1_ref, dv_ref, a_ref, o_ref):
        aggcat = jnp.concatenate([a_ref[0], a_ref[1]],
                                 axis=1).astype(jnp.float32)
        o_ref[...] = jnp.maximum(
            m2_ref[...] + 3.0 * m1_ref[...] - 2.0 * dv_ref[...] * aggcat, 0.0)

    return pl.pallas_call(
        body,
        grid=(N // B,),
        in_specs=[
            pl.BlockSpec((B, D_FEAT), lambda i: (i, 0)),
            pl.BlockSpec((B, D_FEAT), lambda i: (i, 0)),
            pl.BlockSpec((B, 1), lambda i: (i, 0)),
            pl.BlockSpec((NC, B, DH), lambda i: (0, i, 0)),
        ],
        out_specs=pl.BlockSpec((B, D_FEAT), lambda i: (i, 0)),
        out_shape=jax.ShapeDtypeStruct((N, D_FEAT), jnp.float32),
    )(m2, m1, dv, agg2)


def kernel(x, edge_index, W):
    src = edge_index[0].astype(jnp.int32)
    dst = edge_index[1].astype(jnp.int32)

    src_r = src.reshape(NS, E_PER_TILE)
    offs = (jnp.arange(NC, dtype=jnp.int32) * N)[:, None, None]
    src_adj = jnp.concatenate(
        [src_r[None] + offs,
         jnp.zeros((NC, NS, PAD), jnp.int32)],
        axis=2).reshape(NC, NS, NITER, CHUNK)
    src_u = jnp.concatenate(
        [src_r, jnp.zeros((NS, PAD), jnp.int32)],
        axis=1).reshape(NS, NITER, CHUNK)
    dst_p = jnp.concatenate(
        [dst.reshape(NS, E_PER_TILE),
         jnp.full((NS, PAD), N, jnp.int32)],
        axis=1).reshape(NS, NITER, CHUNK)

    m2, tb1 = _tc_matmul(x, W)

    agg1, deg_t = _sc_lap_deg(tb1.reshape(NC * N, DH), src_adj, src_u, dst_p)
    agg1 = agg1.reshape(NC, N, DH)
    deg = deg_t.reshape(NS * HB * 16)[:N].reshape(N, 1)

    m1, t2, dv = _tc_mid(m2, agg1, deg)

    agg2, = _sc_lap(t2.reshape(NC * N, DH), src_adj, src_u, dst_p)
    agg2 = agg2.reshape(NC, N, DH)

    return _tc_final(m2, m1, dv, agg2)


# speedup vs baseline: 1.7111x; 1.2275x over previous
import functools

import jax
import jax.numpy as jnp
from jax import lax
from jax.experimental import pallas as pl
from jax.experimental.pallas import tpu as pltpu
from jax.experimental.pallas import tpu_sc as plsc

N = 10000
E = 320000
D_FEAT = 128
DH = 64
NS = 16
NC = 2
E_PER_TILE = E // NS
CHUNK = 128
NITER = 160
E_PAD = NITER * CHUNK
PAD = E_PAD - E_PER_TILE
STRIPE = N // NS
NRING = 3
_PATH = (0, 1, 1, 1, 0, 1)
LROWS = 640
HB = LROWS // NS


def _make_sc_lap(with_deg):
    mesh = plsc.VectorSubcoreMesh(core_axis_name="c", subcore_axis_name="s")
    out_type = [jax.ShapeDtypeStruct((NC, NS, STRIPE, DH), jnp.bfloat16)]
    if with_deg:
        out_type.append(jax.ShapeDtypeStruct((NS, HB, 16), jnp.float32))

    scratch = [
        pltpu.VMEM((NITER, CHUNK), jnp.int32),
        pltpu.VMEM((NITER, CHUNK), jnp.int32),
        pltpu.VMEM((NITER, CHUNK), jnp.int32),
        pltpu.VMEM((NRING, CHUNK, DH), jnp.bfloat16),
        pltpu.VMEM_SHARED((N + 16, DH), jnp.bfloat16),
        pltpu.VMEM_SHARED((N, DH), jnp.bfloat16),
        [pltpu.SemaphoreType.DMA] * NRING,
        [pltpu.SemaphoreType.DMA] * NRING,
    ]
    if with_deg:
        scratch += [
            pltpu.VMEM((LROWS, 16), jnp.float32),
            pltpu.VMEM((5, CHUNK), jnp.int32),
            pltpu.VMEM_SHARED((LROWS, 16), jnp.float32),
        ]

    @functools.partial(pl.kernel, out_type=out_type, mesh=mesh,
                       compiler_params=pltpu.CompilerParams(
                           use_tc_tiling_on_sc=False,
                           needs_layout_passes=False),
                       scratch_types=scratch)
    def lap(table, src_adj, src_r, dst_r, *refs):
        if with_deg:
            (out, deg_out, sidx, sidx2, didx, bfr, agg, tsp, semg, sems,
             ldeg, idv, degsp) = refs
        else:
            out, sidx, sidx2, didx, bfr, agg, tsp, semg, sems = refs
        c = lax.axis_index("c")
        s = lax.axis_index("s")

        pltpu.sync_copy(src_adj.at[c, s], sidx)
        pltpu.sync_copy(src_r.at[s], sidx2)
        pltpu.sync_copy(dst_r.at[s], didx)
        pltpu.sync_copy(table.at[pl.ds(c * N + s * STRIPE, STRIPE)],
                        tsp.at[pl.ds(s * STRIPE, STRIPE)])

        zv = jnp.zeros((16,), jnp.float32)
        zvb = jnp.zeros((32,), jnp.bfloat16)

        def zrow(i, carry):
            for j in range(DH // 32):
                bfr[0, i, pl.ds(j * 32, 32)] = zvb
            return carry

        lax.fori_loop(0, CHUNK, zrow, 0)
        for k in range(STRIPE // CHUNK):
            pltpu.sync_copy(bfr.at[0],
                            agg.at[pl.ds(s * STRIPE + k * CHUNK, CHUNK)])
        rem = STRIPE % CHUNK
        pltpu.sync_copy(
            bfr.at[0, pl.ds(0, rem)],
            agg.at[pl.ds(s * STRIPE + (STRIPE // CHUNK) * CHUNK, rem)])

        @pl.when(s == 0)
        def _():
            pltpu.sync_copy(bfr.at[0, pl.ds(0, 16)], agg.at[pl.ds(N, 16)])

        if with_deg:
            def zhrow(i, carry):
                ldeg[i, pl.ds(0, 16)] = zv
                return carry

            lax.fori_loop(0, LROWS, zhrow, 0)
            pltpu.sync_copy(ldeg.at[pl.ds(0, HB)],
                            degsp.at[pl.ds(s * HB, HB)])
            iota = lax.iota(jnp.int32, 16)
            for k in range(5):
                for j in range(CHUNK // 16):
                    idv[k, pl.ds(16 * j, 16)] = iota + (CHUNK * k + 16 * j)

        plsc.subcore_barrier()

        def g_start(b, i, sp):
            if sp:
                pltpu.async_copy(tsp.at[sidx2.at[i]], bfr.at[b], semg[b])
            else:
                pltpu.async_copy(table.at[sidx.at[i]], bfr.at[b], semg[b])

        def g_wait(b, i, sp):
            if sp:
                pltpu.make_async_copy(tsp.at[sidx2.at[i]], bfr.at[b],
                                      semg[b]).wait()
            else:
                pltpu.make_async_copy(table.at[sidx.at[i]], bfr.at[b],
                                      semg[b]).wait()

        def s_start(b, i):
            pltpu.async_copy(bfr.at[b], agg.at[didx.at[i]], sems[b],
                             add=True)

        def s_wait(b, i):
            pltpu.make_async_copy(bfr.at[b], agg.at[didx.at[i]],
                                  sems[b]).wait()

        onesv = jnp.ones((16,), jnp.float32)

        def hist(i):
            if with_deg:
                for j in range(CHUNK // 16):
                    nv = didx[i, pl.ds(16 * j, 16)]
                    row = lax.shift_right_logical(nv, 4)
                    col = jnp.bitwise_and(nv, 15)
                    plsc.addupdate_scatter(ldeg, [row, col], onesv)

        def step(i, b, im6, do_swait=True, do_gstart=True):
            par = _PATH[im6]
            hist(i)
            if do_swait:
                s_wait((b + 2) % NRING, i - 1)
            g_wait(b, i, par)
            s_start(b, i)
            if do_gstart:
                g_start((b + 2) % NRING, i + 2, _PATH[(im6 + 2) % 6])

        g_start(0, 0, _PATH[0])
        g_start(1, 1, _PATH[1])
        step(0, 0, 0, do_swait=False)

        def body(j, carry):
            for k in range(2 * NRING):
                i = 1 + 2 * NRING * j + k
                step(i, (1 + k) % NRING, (1 + k) % 6)
            return carry

        lax.fori_loop(0, (NITER - 4) // (2 * NRING), body, 0)

        step(NITER - 3, (NITER - 3) % NRING, (NITER - 3) % 6)
        step(NITER - 2, (NITER - 2) % NRING, (NITER - 2) % 6,
             do_gstart=False)
        step(NITER - 1, (NITER - 1) % NRING, (NITER - 1) % 6,
             do_gstart=False)
        s_wait((NITER - 1) % NRING, NITER - 1)

        if with_deg:
            plsc.subcore_barrier()
            for k in range(5):
                pltpu.sync_copy(ldeg.at[pl.ds(CHUNK * k, CHUNK)],
                                degsp.at[idv.at[k]], add=True)
            plsc.subcore_barrier()

            @pl.when(c == 0)
            def _():
                pltpu.sync_copy(degsp.at[pl.ds(s * HB, HB)], deg_out.at[s])

        plsc.subcore_barrier()
        pltpu.sync_copy(agg.at[pl.ds(s * STRIPE, STRIPE)], out.at[c, s])

    return lap


_sc_lap_deg = _make_sc_lap(True)
_sc_lap = _make_sc_lap(False)


def _tc_matmul(x, W):
    B = 1000

    def body(x_ref, w_ref, m2_ref, tb_ref):
        xv = x_ref[...]
        m2v = jnp.dot(xv, w_ref[...], preferred_element_type=jnp.float32)
        m2_ref[...] = m2v
        tbv = m2v.astype(jnp.bfloat16)
        tb_ref[0] = tbv[:, :DH]
        tb_ref[1] = tbv[:, DH:]

    return pl.pallas_call(
        body,
        grid=(N // B,),
        in_specs=[
            pl.BlockSpec((B, D_FEAT), lambda i: (i, 0)),
            pl.BlockSpec((D_FEAT, D_FEAT), lambda i: (0, 0)),
        ],
        out_specs=[
            pl.BlockSpec((B, D_FEAT), lambda i: (i, 0)),
            pl.BlockSpec((NC, B, DH), lambda i: (0, i, 0)),
        ],
        out_shape=[
            jax.ShapeDtypeStruct((N, D_FEAT), jnp.float32),
            jax.ShapeDtypeStruct((NC, N, DH), jnp.bfloat16),
        ],
    )(x, W)


def _tc_mid(m2, agg1, deg):
    B = 1000

    def body(m2_ref, a_ref, deg_ref, m1_ref, t2_ref, dv_ref):
        m2v = m2_ref[...]
        aggcat = jnp.concatenate([a_ref[0], a_ref[1]],
                                 axis=1).astype(jnp.float32)
        degv = deg_ref[...]
        m1 = degv * m2v - aggcat
        dinv = jnp.where(degv > 0.0, 1.0 / jnp.sqrt(jnp.maximum(degv, 1.0)),
                         0.0)
        hs = m1 * dinv
        hsp = hs.astype(jnp.bfloat16)
        m1_ref[...] = m1
        t2_ref[0] = hsp[:, :DH]
        t2_ref[1] = hsp[:, DH:]
        dv_ref[...] = dinv

    return pl.pallas_call(
        body,
        grid=(N // B,),
        in_specs=[
            pl.BlockSpec((B, D_FEAT), lambda i: (i, 0)),
            pl.BlockSpec((NC, B, DH), lambda i: (0, i, 0)),
            pl.BlockSpec((B, 1), lambda i: (i, 0)),
        ],
        out_specs=[
            pl.BlockSpec((B, D_FEAT), lambda i: (i, 0)),
            pl.BlockSpec((NC, B, DH), lambda i: (0, i, 0)),
            pl.BlockSpec((B, 1), lambda i: (i, 0)),
        ],
        out_shape=[
            jax.ShapeDtypeStruct((N, D_FEAT), jnp.float32),
            jax.ShapeDtypeStruct((NC, N, DH), jnp.bfloat16),
            jax.ShapeDtypeStruct((N, 1), jnp.float32),
        ],
    )(m2, agg1, deg)


def _tc_final(m2, m1, dv, agg2):
    B = 1000

    def body(m2_ref, m1_ref, dv_ref, a_ref, o_ref):
        aggcat = jnp.concatenate([a_ref[0], a_ref[1]],
                                 axis=1).astype(jnp.float32)
        o_ref[...] = jnp.maximum(
            m2_ref[...] + 3.0 * m1_ref[...] - 2.0 * dv_ref[...] * aggcat, 0.0)

    return pl.pallas_call(
        body,
        grid=(N // B,),
        in_specs=[
            pl.BlockSpec((B, D_FEAT), lambda i: (i, 0)),
            pl.BlockSpec((B, D_FEAT), lambda i: (i, 0)),
            pl.BlockSpec((B, 1), lambda i: (i, 0)),
            pl.BlockSpec((NC, B, DH), lambda i: (0, i, 0)),
        ],
        out_specs=pl.BlockSpec((B, D_FEAT), lambda i: (i, 0)),
        out_shape=jax.ShapeDtypeStruct((N, D_FEAT), jnp.float32),
    )(m2, m1, dv, agg2)


def kernel(x, edge_index, W):
    src = edge_index[0].astype(jnp.int32)
    dst = edge_index[1].astype(jnp.int32)

    src_r = src.reshape(NS, E_PER_TILE)
    offs = (jnp.arange(NC, dtype=jnp.int32) * N)[:, None, None]
    src_adj = jnp.concatenate(
        [src_r[None] + offs,
         jnp.zeros((NC, NS, PAD), jnp.int32)],
        axis=2).reshape(NC, NS, NITER, CHUNK)
    src_u = jnp.concatenate(
        [src_r, jnp.zeros((NS, PAD), jnp.int32)],
        axis=1).reshape(NS, NITER, CHUNK)
    dst_p = jnp.concatenate(
        [dst.reshape(NS, E_PER_TILE),
         jnp.full((NS, PAD), N, jnp.int32)],
        axis=1).reshape(NS, NITER, CHUNK)

    m2, tb1 = _tc_matmul(x, W)

    agg1, deg_t = _sc_lap_deg(tb1.reshape(NC * N, DH), src_adj, src_u, dst_p)
    agg1 = agg1.reshape(NC, N, DH)
    deg = deg_t.reshape(NS * HB * 16)[:N].reshape(N, 1)

    m1, t2, dv = _tc_mid(m2, agg1, deg)

    agg2, = _sc_lap(t2.reshape(NC * N, DH), src_adj, src_u, dst_p)
    agg2 = agg2.reshape(NC, N, DH)

    return _tc_final(m2, m1, dv, agg2)


# speedup vs baseline: 1.7472x; 1.0211x over previous
import functools

import jax
import jax.numpy as jnp
from jax import lax
from jax.experimental import pallas as pl
from jax.experimental.pallas import tpu as pltpu
from jax.experimental.pallas import tpu_sc as plsc

N = 10000
E = 320000
D_FEAT = 128
DH = 64
NS = 16
NC = 2
E_PER_TILE = E // NS
CHUNK = 128
NITER = 160
E_PAD = NITER * CHUNK
PAD = E_PAD - E_PER_TILE
STRIPE = N // NS
NRING = 3
_PATH = (0, 1, 1, 1, 1, 1)
LROWS = 640
HB = LROWS // NS


def _make_sc_lap(with_deg):
    mesh = plsc.VectorSubcoreMesh(core_axis_name="c", subcore_axis_name="s")
    out_type = [jax.ShapeDtypeStruct((NC, NS, STRIPE, DH), jnp.bfloat16)]
    if with_deg:
        out_type.append(jax.ShapeDtypeStruct((NS, HB, 16), jnp.float32))

    scratch = [
        pltpu.VMEM((NITER, CHUNK), jnp.int32),
        pltpu.VMEM((NITER, CHUNK), jnp.int32),
        pltpu.VMEM((NITER, CHUNK), jnp.int32),
        pltpu.VMEM((NRING, CHUNK, DH), jnp.bfloat16),
        pltpu.VMEM_SHARED((N + 16, DH), jnp.bfloat16),
        pltpu.VMEM_SHARED((N, DH), jnp.bfloat16),
        [pltpu.SemaphoreType.DMA] * NRING,
        [pltpu.SemaphoreType.DMA] * NRING,
    ]
    if with_deg:
        scratch += [
            pltpu.VMEM((LROWS, 16), jnp.float32),
            pltpu.VMEM((5, CHUNK), jnp.int32),
            pltpu.VMEM_SHARED((LROWS, 16), jnp.float32),
        ]

    @functools.partial(pl.kernel, out_type=out_type, mesh=mesh,
                       compiler_params=pltpu.CompilerParams(
                           use_tc_tiling_on_sc=False,
                           needs_layout_passes=False),
                       scratch_types=scratch)
    def lap(table, src_adj, src_r, dst_r, *refs):
        if with_deg:
            (out, deg_out, sidx, sidx2, didx, bfr, agg, tsp, semg, sems,
             ldeg, idv, degsp) = refs
        else:
            out, sidx, sidx2, didx, bfr, agg, tsp, semg, sems = refs
        c = lax.axis_index("c")
        s = lax.axis_index("s")

        pltpu.sync_copy(src_adj.at[c, s], sidx)
        pltpu.sync_copy(src_r.at[s], sidx2)
        pltpu.sync_copy(dst_r.at[s], didx)
        pltpu.sync_copy(table.at[pl.ds(c * N + s * STRIPE, STRIPE)],
                        tsp.at[pl.ds(s * STRIPE, STRIPE)])

        zv = jnp.zeros((16,), jnp.float32)
        zvb = jnp.zeros((32,), jnp.bfloat16)

        def zrow(i, carry):
            for j in range(DH // 32):
                bfr[0, i, pl.ds(j * 32, 32)] = zvb
            return carry

        lax.fori_loop(0, CHUNK, zrow, 0)
        for k in range(STRIPE // CHUNK):
            pltpu.sync_copy(bfr.at[0],
                            agg.at[pl.ds(s * STRIPE + k * CHUNK, CHUNK)])
        rem = STRIPE % CHUNK
        pltpu.sync_copy(
            bfr.at[0, pl.ds(0, rem)],
            agg.at[pl.ds(s * STRIPE + (STRIPE // CHUNK) * CHUNK, rem)])

        @pl.when(s == 0)
        def _():
            pltpu.sync_copy(bfr.at[0, pl.ds(0, 16)], agg.at[pl.ds(N, 16)])

        if with_deg:
            def zhrow(i, carry):
                ldeg[i, pl.ds(0, 16)] = zv
                return carry

            lax.fori_loop(0, LROWS, zhrow, 0)
            pltpu.sync_copy(ldeg.at[pl.ds(0, HB)],
                            degsp.at[pl.ds(s * HB, HB)])
            iota = lax.iota(jnp.int32, 16)
            for k in range(5):
                for j in range(CHUNK // 16):
                    idv[k, pl.ds(16 * j, 16)] = iota + (CHUNK * k + 16 * j)

        plsc.subcore_barrier()

        def g_start(b, i, sp):
            if sp:
                pltpu.async_copy(tsp.at[sidx2.at[i]], bfr.at[b], semg[b])
            else:
                pltpu.async_copy(table.at[sidx.at[i]], bfr.at[b], semg[b])

        def g_wait(b, i, sp):
            if sp:
                pltpu.make_async_copy(tsp.at[sidx2.at[i]], bfr.at[b],
                                      semg[b]).wait()
            else:
                pltpu.make_async_copy(table.at[sidx.at[i]], bfr.at[b],
                                      semg[b]).wait()

        def s_start(b, i):
            pltpu.async_copy(bfr.at[b], agg.at[didx.at[i]], sems[b],
                             add=True)

        def s_wait(b, i):
            pltpu.make_async_copy(bfr.at[b], agg.at[didx.at[i]],
                                  sems[b]).wait()

        onesv = jnp.ones((16,), jnp.float32)

        def hist(i):
            if with_deg:
                for j in range(CHUNK // 16):
                    nv = didx[i, pl.ds(16 * j, 16)]
                    row = lax.shift_right_logical(nv, 4)
                    col = jnp.bitwise_and(nv, 15)
                    plsc.addupdate_scatter(ldeg, [row, col], onesv)

        def step(i, b, im6, do_swait=True, do_gstart=True):
            par = _PATH[im6]
            hist(i)
            if do_swait:
                s_wait((b + 2) % NRING, i - 1)
            g_wait(b, i, par)
            s_start(b, i)
            if do_gstart:
                g_start((b + 2) % NRING, i + 2, _PATH[(im6 + 2) % 6])

        g_start(0, 0, _PATH[0])
        g_start(1, 1, _PATH[1])
        step(0, 0, 0, do_swait=False)

        def body(j, carry):
            for k in range(2 * NRING):
                i = 1 + 2 * NRING * j + k
                step(i, (1 + k) % NRING, (1 + k) % 6)
            return carry

        lax.fori_loop(0, (NITER - 4) // (2 * NRING), body, 0)

        step(NITER - 3, (NITER - 3) % NRING, (NITER - 3) % 6)
        step(NITER - 2, (NITER - 2) % NRING, (NITER - 2) % 6,
             do_gstart=False)
        step(NITER - 1, (NITER - 1) % NRING, (NITER - 1) % 6,
             do_gstart=False)
        s_wait((NITER - 1) % NRING, NITER - 1)

        if with_deg:
            plsc.subcore_barrier()
            for k in range(5):
                pltpu.sync_copy(ldeg.at[pl.ds(CHUNK * k, CHUNK)],
                                degsp.at[idv.at[k]], add=True)
            plsc.subcore_barrier()

            @pl.when(c == 0)
            def _():
                pltpu.sync_copy(degsp.at[pl.ds(s * HB, HB)], deg_out.at[s])

        plsc.subcore_barrier()
        pltpu.sync_copy(agg.at[pl.ds(s * STRIPE, STRIPE)], out.at[c, s])

    return lap


_sc_lap_deg = _make_sc_lap(True)
_sc_lap = _make_sc_lap(False)


def _tc_matmul(x, W):
    B = 1000

    def body(x_ref, w_ref, m2_ref, tb_ref):
        xv = x_ref[...]
        m2v = jnp.dot(xv, w_ref[...], preferred_element_type=jnp.float32)
        m2_ref[...] = m2v
        tbv = m2v.astype(jnp.bfloat16)
        tb_ref[0] = tbv[:, :DH]
        tb_ref[1] = tbv[:, DH:]

    return pl.pallas_call(
        body,
        grid=(N // B,),
        in_specs=[
            pl.BlockSpec((B, D_FEAT), lambda i: (i, 0)),
            pl.BlockSpec((D_FEAT, D_FEAT), lambda i: (0, 0)),
        ],
        out_specs=[
            pl.BlockSpec((B, D_FEAT), lambda i: (i, 0)),
            pl.BlockSpec((NC, B, DH), lambda i: (0, i, 0)),
        ],
        out_shape=[
            jax.ShapeDtypeStruct((N, D_FEAT), jnp.float32),
            jax.ShapeDtypeStruct((NC, N, DH), jnp.bfloat16),
        ],
    )(x, W)


def _tc_mid(m2, agg1, deg):
    B = 1000

    def body(m2_ref, a_ref, deg_ref, m1_ref, t2_ref, dv_ref):
        m2v = m2_ref[...]
        aggcat = jnp.concatenate([a_ref[0], a_ref[1]],
                                 axis=1).astype(jnp.float32)
        degv = deg_ref[...]
        m1 = degv * m2v - aggcat
        dinv = jnp.where(degv > 0.0, 1.0 / jnp.sqrt(jnp.maximum(degv, 1.0)),
                         0.0)
        hs = m1 * dinv
        hsp = hs.astype(jnp.bfloat16)
        m1_ref[...] = m1
        t2_ref[0] = hsp[:, :DH]
        t2_ref[1] = hsp[:, DH:]
        dv_ref[...] = dinv

    return pl.pallas_call(
        body,
        grid=(N // B,),
        in_specs=[
            pl.BlockSpec((B, D_FEAT), lambda i: (i, 0)),
            pl.BlockSpec((NC, B, DH), lambda i: (0, i, 0)),
            pl.BlockSpec((B, 1), lambda i: (i, 0)),
        ],
        out_specs=[
            pl.BlockSpec((B, D_FEAT), lambda i: (i, 0)),
            pl.BlockSpec((NC, B, DH), lambda i: (0, i, 0)),
            pl.BlockSpec((B, 1), lambda i: (i, 0)),
        ],
        out_shape=[
            jax.ShapeDtypeStruct((N, D_FEAT), jnp.float32),
            jax.ShapeDtypeStruct((NC, N, DH), jnp.bfloat16),
            jax.ShapeDtypeStruct((N, 1), jnp.float32),
        ],
    )(m2, agg1, deg)


def _tc_final(m2, m1, dv, agg2):
    B = 1000

    def body(m2_ref, m1_ref, dv_ref, a_ref, o_ref):
        aggcat = jnp.concatenate([a_ref[0], a_ref[1]],
                                 axis=1).astype(jnp.float32)
        o_ref[...] = jnp.maximum(
            m2_ref[...] + 3.0 * m1_ref[...] - 2.0 * dv_ref[...] * aggcat, 0.0)

    return pl.pallas_call(
        body,
        grid=(N // B,),
        in_specs=[
            pl.BlockSpec((B, D_FEAT), lambda i: (i, 0)),
            pl.BlockSpec((B, D_FEAT), lambda i: (i, 0)),
            pl.BlockSpec((B, 1), lambda i: (i, 0)),
            pl.BlockSpec((NC, B, DH), lambda i: (0, i, 0)),
        ],
        out_specs=pl.BlockSpec((B, D_FEAT), lambda i: (i, 0)),
        out_shape=jax.ShapeDtypeStruct((N, D_FEAT), jnp.float32),
    )(m2, m1, dv, agg2)


def kernel(x, edge_index, W):
    src = edge_index[0].astype(jnp.int32)
    dst = edge_index[1].astype(jnp.int32)

    src_r = src.reshape(NS, E_PER_TILE)
    offs = (jnp.arange(NC, dtype=jnp.int32) * N)[:, None, None]
    src_adj = jnp.concatenate(
        [src_r[None] + offs,
         jnp.zeros((NC, NS, PAD), jnp.int32)],
        axis=2).reshape(NC, NS, NITER, CHUNK)
    src_u = jnp.concatenate(
        [src_r, jnp.zeros((NS, PAD), jnp.int32)],
        axis=1).reshape(NS, NITER, CHUNK)
    dst_p = jnp.concatenate(
        [dst.reshape(NS, E_PER_TILE),
         jnp.full((NS, PAD), N, jnp.int32)],
        axis=1).reshape(NS, NITER, CHUNK)

    m2, tb1 = _tc_matmul(x, W)

    agg1, deg_t = _sc_lap_deg(tb1.reshape(NC * N, DH), src_adj, src_u, dst_p)
    agg1 = agg1.reshape(NC, N, DH)
    deg = deg_t.reshape(NS * HB * 16)[:N].reshape(N, 1)

    m1, t2, dv = _tc_mid(m2, agg1, deg)

    agg2, = _sc_lap(t2.reshape(NC * N, DH), src_adj, src_u, dst_p)
    agg2 = agg2.reshape(NC, N, DH)

    return _tc_final(m2, m1, dv, agg2)


# speedup vs baseline: 1.9797x; 1.1331x over previous
import functools

import jax
import jax.numpy as jnp
from jax import lax
from jax.experimental import pallas as pl
from jax.experimental.pallas import tpu as pltpu
from jax.experimental.pallas import tpu_sc as plsc

N = 10000
E = 320000
D_FEAT = 128
DH = 64
NS = 16
NC = 2
E_PER_TILE = E // NS
CHUNK = 128
NITER = 160
E_PAD = NITER * CHUNK
PAD = E_PAD - E_PER_TILE
STRIPE = N // NS
NRING = 3
_PATH = (1, 1, 1, 1, 1, 1)
LROWS = 640
HB = LROWS // NS


def _make_sc_lap(with_deg):
    mesh = plsc.VectorSubcoreMesh(core_axis_name="c", subcore_axis_name="s")
    out_type = [jax.ShapeDtypeStruct((NC, NS, STRIPE, DH), jnp.bfloat16)]
    if with_deg:
        out_type.append(jax.ShapeDtypeStruct((NS, HB, 16), jnp.float32))

    scratch = [
        pltpu.VMEM((NITER, CHUNK), jnp.int32),
        pltpu.VMEM((NITER, CHUNK), jnp.int32),
        pltpu.VMEM((NITER, CHUNK), jnp.int32),
        pltpu.VMEM((NRING, CHUNK, DH), jnp.bfloat16),
        pltpu.VMEM_SHARED((N + 16, DH), jnp.bfloat16),
        pltpu.VMEM_SHARED((N, DH), jnp.bfloat16),
        [pltpu.SemaphoreType.DMA] * NRING,
        [pltpu.SemaphoreType.DMA] * NRING,
    ]
    if with_deg:
        scratch += [
            pltpu.VMEM((LROWS, 16), jnp.float32),
            pltpu.VMEM((5, CHUNK), jnp.int32),
            pltpu.VMEM_SHARED((LROWS, 16), jnp.float32),
        ]

    @functools.partial(pl.kernel, out_type=out_type, mesh=mesh,
                       compiler_params=pltpu.CompilerParams(
                           use_tc_tiling_on_sc=False,
                           needs_layout_passes=False),
                       scratch_types=scratch)
    def lap(table, src_adj, src_r, dst_r, *refs):
        if with_deg:
            (out, deg_out, sidx, sidx2, didx, bfr, agg, tsp, semg, sems,
             ldeg, idv, degsp) = refs
        else:
            out, sidx, sidx2, didx, bfr, agg, tsp, semg, sems = refs
        c = lax.axis_index("c")
        s = lax.axis_index("s")

        pltpu.sync_copy(src_adj.at[c, s], sidx)
        pltpu.sync_copy(src_r.at[s], sidx2)
        pltpu.sync_copy(dst_r.at[s], didx)
        pltpu.sync_copy(table.at[pl.ds(c * N + s * STRIPE, STRIPE)],
                        tsp.at[pl.ds(s * STRIPE, STRIPE)])

        zv = jnp.zeros((16,), jnp.float32)
        zvb = jnp.zeros((32,), jnp.bfloat16)

        def zrow(i, carry):
            for j in range(DH // 32):
                bfr[0, i, pl.ds(j * 32, 32)] = zvb
            return carry

        lax.fori_loop(0, CHUNK, zrow, 0)
        for k in range(STRIPE // CHUNK):
            pltpu.sync_copy(bfr.at[0],
                            agg.at[pl.ds(s * STRIPE + k * CHUNK, CHUNK)])
        rem = STRIPE % CHUNK
        pltpu.sync_copy(
            bfr.at[0, pl.ds(0, rem)],
            agg.at[pl.ds(s * STRIPE + (STRIPE // CHUNK) * CHUNK, rem)])

        @pl.when(s == 0)
        def _():
            pltpu.sync_copy(bfr.at[0, pl.ds(0, 16)], agg.at[pl.ds(N, 16)])

        if with_deg:
            def zhrow(i, carry):
                ldeg[i, pl.ds(0, 16)] = zv
                return carry

            lax.fori_loop(0, LROWS, zhrow, 0)
            pltpu.sync_copy(ldeg.at[pl.ds(0, HB)],
                            degsp.at[pl.ds(s * HB, HB)])
            iota = lax.iota(jnp.int32, 16)
            for k in range(5):
                for j in range(CHUNK // 16):
                    idv[k, pl.ds(16 * j, 16)] = iota + (CHUNK * k + 16 * j)

        plsc.subcore_barrier()

        def g_start(b, i, sp):
            if sp:
                pltpu.async_copy(tsp.at[sidx2.at[i]], bfr.at[b], semg[b])
            else:
                pltpu.async_copy(table.at[sidx.at[i]], bfr.at[b], semg[b])

        def g_wait(b, i, sp):
            if sp:
                pltpu.make_async_copy(tsp.at[sidx2.at[i]], bfr.at[b],
                                      semg[b]).wait()
            else:
                pltpu.make_async_copy(table.at[sidx.at[i]], bfr.at[b],
                                      semg[b]).wait()

        def s_start(b, i):
            pltpu.async_copy(bfr.at[b], agg.at[didx.at[i]], sems[b],
                             add=True)

        def s_wait(b, i):
            pltpu.make_async_copy(bfr.at[b], agg.at[didx.at[i]],
                                  sems[b]).wait()

        onesv = jnp.ones((16,), jnp.float32)

        def hist(i):
            if with_deg:
                for j in range(CHUNK // 16):
                    nv = didx[i, pl.ds(16 * j, 16)]
                    row = lax.shift_right_logical(nv, 4)
                    col = jnp.bitwise_and(nv, 15)
                    plsc.addupdate_scatter(ldeg, [row, col], onesv)

        def step(i, b, im6, do_swait=True, do_gstart=True):
            par = _PATH[im6]
            hist(i)
            if do_swait:
                s_wait((b + 2) % NRING, i - 1)
            g_wait(b, i, par)
            s_start(b, i)
            if do_gstart:
                g_start((b + 2) % NRING, i + 2, _PATH[(im6 + 2) % 6])

        g_start(0, 0, _PATH[0])
        g_start(1, 1, _PATH[1])
        step(0, 0, 0, do_swait=False)

        def body(j, carry):
            for k in range(2 * NRING):
                i = 1 + 2 * NRING * j + k
                step(i, (1 + k) % NRING, (1 + k) % 6)
            return carry

        lax.fori_loop(0, (NITER - 4) // (2 * NRING), body, 0)

        step(NITER - 3, (NITER - 3) % NRING, (NITER - 3) % 6)
        step(NITER - 2, (NITER - 2) % NRING, (NITER - 2) % 6,
             do_gstart=False)
        step(NITER - 1, (NITER - 1) % NRING, (NITER - 1) % 6,
             do_gstart=False)
        s_wait((NITER - 1) % NRING, NITER - 1)

        if with_deg:
            plsc.subcore_barrier()
            for k in range(5):
                pltpu.sync_copy(ldeg.at[pl.ds(CHUNK * k, CHUNK)],
                                degsp.at[idv.at[k]], add=True)
            plsc.subcore_barrier()

            @pl.when(c == 0)
            def _():
                pltpu.sync_copy(degsp.at[pl.ds(s * HB, HB)], deg_out.at[s])

        plsc.subcore_barrier()
        pltpu.sync_copy(agg.at[pl.ds(s * STRIPE, STRIPE)], out.at[c, s])

    return lap


_sc_lap_deg = _make_sc_lap(True)
_sc_lap = _make_sc_lap(False)


def _tc_matmul(x, W):
    B = 1000

    def body(x_ref, w_ref, m2_ref, tb_ref):
        xv = x_ref[...]
        m2v = jnp.dot(xv, w_ref[...], preferred_element_type=jnp.float32)
        m2_ref[...] = m2v
        tbv = m2v.astype(jnp.bfloat16)
        tb_ref[0] = tbv[:, :DH]
        tb_ref[1] = tbv[:, DH:]

    return pl.pallas_call(
        body,
        grid=(N // B,),
        in_specs=[
            pl.BlockSpec((B, D_FEAT), lambda i: (i, 0)),
            pl.BlockSpec((D_FEAT, D_FEAT), lambda i: (0, 0)),
        ],
        out_specs=[
            pl.BlockSpec((B, D_FEAT), lambda i: (i, 0)),
            pl.BlockSpec((NC, B, DH), lambda i: (0, i, 0)),
        ],
        out_shape=[
            jax.ShapeDtypeStruct((N, D_FEAT), jnp.float32),
            jax.ShapeDtypeStruct((NC, N, DH), jnp.bfloat16),
        ],
    )(x, W)


def _tc_mid(m2, agg1, deg):
    B = 1000

    def body(m2_ref, a_ref, deg_ref, m1_ref, t2_ref, dv_ref):
        m2v = m2_ref[...]
        aggcat = jnp.concatenate([a_ref[0], a_ref[1]],
                                 axis=1).astype(jnp.float32)
        degv = deg_ref[...]
        m1 = degv * m2v - aggcat
        dinv = jnp.where(degv > 0.0, 1.0 / jnp.sqrt(jnp.maximum(degv, 1.0)),
                         0.0)
        hs = m1 * dinv
        hsp = hs.astype(jnp.bfloat16)
        m1_ref[...] = m1
        t2_ref[0] = hsp[:, :DH]
        t2_ref[1] = hsp[:, DH:]
        dv_ref[...] = dinv

    return pl.pallas_call(
        body,
        grid=(N // B,),
        in_specs=[
            pl.BlockSpec((B, D_FEAT), lambda i: (i, 0)),
            pl.BlockSpec((NC, B, DH), lambda i: (0, i, 0)),
            pl.BlockSpec((B, 1), lambda i: (i, 0)),
        ],
        out_specs=[
            pl.BlockSpec((B, D_FEAT), lambda i: (i, 0)),
            pl.BlockSpec((NC, B, DH), lambda i: (0, i, 0)),
            pl.BlockSpec((B, 1), lambda i: (i, 0)),
        ],
        out_shape=[
            jax.ShapeDtypeStruct((N, D_FEAT), jnp.float32),
            jax.ShapeDtypeStruct((NC, N, DH), jnp.bfloat16),
            jax.ShapeDtypeStruct((N, 1), jnp.float32),
        ],
    )(m2, agg1, deg)


def _tc_final(m2, m1, dv, agg2):
    B = 1000

    def body(m2_ref, m1_ref, dv_ref, a_ref, o_ref):
        aggcat = jnp.concatenate([a_ref[0], a_ref[1]],
                                 axis=1).astype(jnp.float32)
        o_ref[...] = jnp.maximum(
            m2_ref[...] + 3.0 * m1_ref[...] - 2.0 * dv_ref[...] * aggcat, 0.0)

    return pl.pallas_call(
        body,
        grid=(N // B,),
        in_specs=[
            pl.BlockSpec((B, D_FEAT), lambda i: (i, 0)),
            pl.BlockSpec((B, D_FEAT), lambda i: (i, 0)),
            pl.BlockSpec((B, 1), lambda i: (i, 0)),
            pl.BlockSpec((NC, B, DH), lambda i: (0, i, 0)),
        ],
        out_specs=pl.BlockSpec((B, D_FEAT), lambda i: (i, 0)),
        out_shape=jax.ShapeDtypeStruct((N, D_FEAT), jnp.float32),
    )(m2, m1, dv, agg2)


def kernel(x, edge_index, W):
    src = edge_index[0].astype(jnp.int32)
    dst = edge_index[1].astype(jnp.int32)

    src_r = src.reshape(NS, E_PER_TILE)
    offs = (jnp.arange(NC, dtype=jnp.int32) * N)[:, None, None]
    src_adj = jnp.concatenate(
        [src_r[None] + offs,
         jnp.zeros((NC, NS, PAD), jnp.int32)],
        axis=2).reshape(NC, NS, NITER, CHUNK)
    src_u = jnp.concatenate(
        [src_r, jnp.zeros((NS, PAD), jnp.int32)],
        axis=1).reshape(NS, NITER, CHUNK)
    dst_p = jnp.concatenate(
        [dst.reshape(NS, E_PER_TILE),
         jnp.full((NS, PAD), N, jnp.int32)],
        axis=1).reshape(NS, NITER, CHUNK)

    m2, tb1 = _tc_matmul(x, W)

    agg1, deg_t = _sc_lap_deg(tb1.reshape(NC * N, DH), src_adj, src_u, dst_p)
    agg1 = agg1.reshape(NC, N, DH)
    deg = deg_t.reshape(NS * HB * 16)[:N].reshape(N, 1)

    m1, t2, dv = _tc_mid(m2, agg1, deg)

    agg2, = _sc_lap(t2.reshape(NC * N, DH), src_adj, src_u, dst_p)
    agg2 = agg2.reshape(NC, N, DH)

    return _tc_final(m2, m1, dv, agg2)


# speedup vs baseline: 2.0116x; 1.0161x over previous
import functools

import jax
import jax.numpy as jnp
from jax import lax
from jax.experimental import pallas as pl
from jax.experimental.pallas import tpu as pltpu
from jax.experimental.pallas import tpu_sc as plsc

N = 10000
E = 320000
D_FEAT = 128
DH = 64
NS = 16
NC = 2
E_PER_TILE = E // NS
CHUNK = 128
NITER = 160
E_PAD = NITER * CHUNK
PAD = E_PAD - E_PER_TILE
STRIPE = N // NS
NRING = 3
LROWS = 640
HB = LROWS // NS


def _make_sc_lap(with_deg):
    mesh = plsc.VectorSubcoreMesh(core_axis_name="c", subcore_axis_name="s")
    out_type = [jax.ShapeDtypeStruct((NC, NS, STRIPE, DH), jnp.bfloat16)]
    if with_deg:
        out_type.append(jax.ShapeDtypeStruct((NS, HB, 16), jnp.float32))

    scratch = [
        pltpu.VMEM((NITER, CHUNK), jnp.int32),
        pltpu.VMEM((NITER, CHUNK), jnp.int32),
        pltpu.VMEM((NRING, CHUNK, DH), jnp.bfloat16),
        pltpu.VMEM_SHARED((N + 16, DH), jnp.bfloat16),
        pltpu.VMEM_SHARED((N, DH), jnp.bfloat16),
        [pltpu.SemaphoreType.DMA] * NRING,
        [pltpu.SemaphoreType.DMA] * NRING,
    ]
    if with_deg:
        scratch += [
            pltpu.VMEM((LROWS, 16), jnp.float32),
            pltpu.VMEM((5, CHUNK), jnp.int32),
            pltpu.VMEM_SHARED((LROWS, 16), jnp.float32),
        ]

    @functools.partial(pl.kernel, out_type=out_type, mesh=mesh,
                       compiler_params=pltpu.CompilerParams(
                           use_tc_tiling_on_sc=False,
                           needs_layout_passes=False),
                       scratch_types=scratch)
    def lap(table, src_r, dst_r, *refs):
        if with_deg:
            (out, deg_out, sidx, didx, bfr, agg, tsp, semg, sems,
             ldeg, idv, degsp) = refs
        else:
            out, sidx, didx, bfr, agg, tsp, semg, sems = refs
        c = lax.axis_index("c")
        s = lax.axis_index("s")

        pltpu.sync_copy(src_r.at[s], sidx)
        pltpu.sync_copy(dst_r.at[s], didx)
        pltpu.sync_copy(table.at[pl.ds(c * N + s * STRIPE, STRIPE)],
                        tsp.at[pl.ds(s * STRIPE, STRIPE)])

        zv = jnp.zeros((16,), jnp.float32)
        zvb = jnp.zeros((32,), jnp.bfloat16)

        def zrow(i, carry):
            for j in range(DH // 32):
                bfr[0, i, pl.ds(j * 32, 32)] = zvb
            return carry

        lax.fori_loop(0, CHUNK, zrow, 0)
        for k in range(STRIPE // CHUNK):
            pltpu.sync_copy(bfr.at[0],
                            agg.at[pl.ds(s * STRIPE + k * CHUNK, CHUNK)])
        rem = STRIPE % CHUNK
        pltpu.sync_copy(
            bfr.at[0, pl.ds(0, rem)],
            agg.at[pl.ds(s * STRIPE + (STRIPE // CHUNK) * CHUNK, rem)])

        @pl.when(s == 0)
        def _():
            pltpu.sync_copy(bfr.at[0, pl.ds(0, 16)], agg.at[pl.ds(N, 16)])

        if with_deg:
            def zhrow(i, carry):
                ldeg[i, pl.ds(0, 16)] = zv
                return carry

            lax.fori_loop(0, LROWS, zhrow, 0)
            pltpu.sync_copy(ldeg.at[pl.ds(0, HB)],
                            degsp.at[pl.ds(s * HB, HB)])
            iota = lax.iota(jnp.int32, 16)
            for k in range(5):
                for j in range(CHUNK // 16):
                    idv[k, pl.ds(16 * j, 16)] = iota + (CHUNK * k + 16 * j)

        plsc.subcore_barrier()

        def g_start(b, i):
            pltpu.async_copy(tsp.at[sidx.at[i]], bfr.at[b], semg[b])

        def g_wait(b, i):
            pltpu.make_async_copy(tsp.at[sidx.at[i]], bfr.at[b],
                                  semg[b]).wait()

        def s_start(b, i):
            pltpu.async_copy(bfr.at[b], agg.at[didx.at[i]], sems[b],
                             add=True)

        def s_wait(b, i):
            pltpu.make_async_copy(bfr.at[b], agg.at[didx.at[i]],
                                  sems[b]).wait()

        onesv = jnp.ones((16,), jnp.float32)

        def hist(i):
            if with_deg:
                for j in range(CHUNK // 16):
                    nv = didx[i, pl.ds(16 * j, 16)]
                    row = lax.shift_right_logical(nv, 4)
                    col = jnp.bitwise_and(nv, 15)
                    plsc.addupdate_scatter(ldeg, [row, col], onesv)

        def step(i, b, do_swait=True, do_gstart=True):
            hist(i)
            if do_swait:
                s_wait((b + 2) % NRING, i - 1)
            g_wait(b, i)
            s_start(b, i)
            if do_gstart:
                g_start((b + 2) % NRING, i + 2)

        g_start(0, 0)
        g_start(1, 1)
        step(0, 0, do_swait=False)

        def body(j, carry):
            for k in range(NRING):
                i = 1 + NRING * j + k
                step(i, (1 + k) % NRING)
            return carry

        lax.fori_loop(0, (NITER - 4) // NRING, body, 0)

        step(NITER - 3, (NITER - 3) % NRING)
        step(NITER - 2, (NITER - 2) % NRING, do_gstart=False)
        step(NITER - 1, (NITER - 1) % NRING, do_gstart=False)
        s_wait((NITER - 1) % NRING, NITER - 1)

        if with_deg:
            plsc.subcore_barrier()
            for k in range(5):
                pltpu.sync_copy(ldeg.at[pl.ds(CHUNK * k, CHUNK)],
                                degsp.at[idv.at[k]], add=True)
            plsc.subcore_barrier()

            @pl.when(c == 0)
            def _():
                pltpu.sync_copy(degsp.at[pl.ds(s * HB, HB)], deg_out.at[s])

        plsc.subcore_barrier()
        pltpu.sync_copy(agg.at[pl.ds(s * STRIPE, STRIPE)], out.at[c, s])

    return lap


_sc_lap_deg = _make_sc_lap(True)
_sc_lap = _make_sc_lap(False)


def _tc_matmul(x, W):
    B = 1000

    def body(x_ref, w_ref, m2_ref, tb_ref):
        xv = x_ref[...]
        m2v = jnp.dot(xv, w_ref[...], preferred_element_type=jnp.float32)
        m2_ref[...] = m2v
        tbv = m2v.astype(jnp.bfloat16)
        tb_ref[0] = tbv[:, :DH]
        tb_ref[1] = tbv[:, DH:]

    return pl.pallas_call(
        body,
        grid=(N // B,),
        in_specs=[
            pl.BlockSpec((B, D_FEAT), lambda i: (i, 0)),
            pl.BlockSpec((D_FEAT, D_FEAT), lambda i: (0, 0)),
        ],
        out_specs=[
            pl.BlockSpec((B, D_FEAT), lambda i: (i, 0)),
            pl.BlockSpec((NC, B, DH), lambda i: (0, i, 0)),
        ],
        out_shape=[
            jax.ShapeDtypeStruct((N, D_FEAT), jnp.float32),
            jax.ShapeDtypeStruct((NC, N, DH), jnp.bfloat16),
        ],
    )(x, W)


def _tc_mid(m2, agg1, deg):
    B = 1000

    def body(m2_ref, a_ref, deg_ref, m1_ref, t2_ref, dv_ref):
        m2v = m2_ref[...]
        aggcat = jnp.concatenate([a_ref[0], a_ref[1]],
                                 axis=1).astype(jnp.float32)
        degv = deg_ref[...]
        m1 = degv * m2v - aggcat
        dinv = jnp.where(degv > 0.0, 1.0 / jnp.sqrt(jnp.maximum(degv, 1.0)),
                         0.0)
        hs = m1 * dinv
        hsp = hs.astype(jnp.bfloat16)
        m1_ref[...] = m1
        t2_ref[0] = hsp[:, :DH]
        t2_ref[1] = hsp[:, DH:]
        dv_ref[...] = dinv

    return pl.pallas_call(
        body,
        grid=(N // B,),
        in_specs=[
            pl.BlockSpec((B, D_FEAT), lambda i: (i, 0)),
            pl.BlockSpec((NC, B, DH), lambda i: (0, i, 0)),
            pl.BlockSpec((B, 1), lambda i: (i, 0)),
        ],
        out_specs=[
            pl.BlockSpec((B, D_FEAT), lambda i: (i, 0)),
            pl.BlockSpec((NC, B, DH), lambda i: (0, i, 0)),
            pl.BlockSpec((B, 1), lambda i: (i, 0)),
        ],
        out_shape=[
            jax.ShapeDtypeStruct((N, D_FEAT), jnp.float32),
            jax.ShapeDtypeStruct((NC, N, DH), jnp.bfloat16),
            jax.ShapeDtypeStruct((N, 1), jnp.float32),
        ],
    )(m2, agg1, deg)


def _tc_final(m2, m1, dv, agg2):
    B = 1000

    def body(m2_ref, m1_ref, dv_ref, a_ref, o_ref):
        aggcat = jnp.concatenate([a_ref[0], a_ref[1]],
                                 axis=1).astype(jnp.float32)
        o_ref[...] = jnp.maximum(
            m2_ref[...] + 3.0 * m1_ref[...] - 2.0 * dv_ref[...] * aggcat, 0.0)

    return pl.pallas_call(
        body,
        grid=(N // B,),
        in_specs=[
            pl.BlockSpec((B, D_FEAT), lambda i: (i, 0)),
            pl.BlockSpec((B, D_FEAT), lambda i: (i, 0)),
            pl.BlockSpec((B, 1), lambda i: (i, 0)),
            pl.BlockSpec((NC, B, DH), lambda i: (0, i, 0)),
        ],
        out_specs=pl.BlockSpec((B, D_FEAT), lambda i: (i, 0)),
        out_shape=jax.ShapeDtypeStruct((N, D_FEAT), jnp.float32),
    )(m2, m1, dv, agg2)


def kernel(x, edge_index, W):
    src = edge_index[0].astype(jnp.int32)
    dst = edge_index[1].astype(jnp.int32)

    src_u = jnp.concatenate(
        [src.reshape(NS, E_PER_TILE),
         jnp.zeros((NS, PAD), jnp.int32)],
        axis=1).reshape(NS, NITER, CHUNK)
    dst_p = jnp.concatenate(
        [dst.reshape(NS, E_PER_TILE),
         jnp.full((NS, PAD), N, jnp.int32)],
        axis=1).reshape(NS, NITER, CHUNK)

    m2, tb1 = _tc_matmul(x, W)

    agg1, deg_t = _sc_lap_deg(tb1.reshape(NC * N, DH), src_u, dst_p)
    agg1 = agg1.reshape(NC, N, DH)
    deg = deg_t.reshape(NS * HB * 16)[:N].reshape(N, 1)

    m1, t2, dv = _tc_mid(m2, agg1, deg)

    agg2, = _sc_lap(t2.reshape(NC * N, DH), src_u, dst_p)
    agg2 = agg2.reshape(NC, N, DH)

    return _tc_final(m2, m1, dv, agg2)
